# Initial kernel scaffold; baseline (speedup 1.0000x reference)
import jax
import jax.numpy as jnp
from jax.experimental import pallas as pl


def kernel(x, pos, edge_index, edge_weight, h_W0, h_b0, h_W1, h_b1, f_W0, f_b0, f_W1, f_b1, g_W0, g_b0, g_W1, g_b1, bn_g, bn_b):
    raise NotImplementedError("write your pallas kernel here")


# speedup vs baseline: 1.1137x; 1.1137x over previous
import functools

import jax
import jax.numpy as jnp
from jax import lax
from jax.experimental import pallas as pl
from jax.experimental.pallas import tpu as pltpu
from jax.experimental.pallas import tpu_sc as plsc

N = 10000
E = 160000
D = 256
H = 512

_NC = 2
_NS = 16
_NW = _NC * _NS

_OWN = 313
_NPAD = _NW * _OWN
_OWNP = 320
_HH = H // 2
_NEG = -3.0e38
_EPAD = E + _NW * 64

_BN1 = 1000


def _stage1_body(x_ref, pos8_ref, hW0_ref, hb0_ref, hW18_ref, hb18_ref,
                 W0x_ref, W0p8_ref, fb0_ref, S_ref, Q_ref):
    x = x_ref[...]
    pos8 = pos8_ref[...]
    hh = jnp.maximum(
        jnp.dot(x, hW0_ref[...], preferred_element_type=jnp.float32)
        + hb0_ref[...], 0.0)
    delta8 = jnp.dot(hh, hW18_ref[...], preferred_element_type=jnp.float32) \
        + hb18_ref[...]
    W0p8 = W0p8_ref[...]
    S_ref[...] = (jnp.dot(x, W0x_ref[...], preferred_element_type=jnp.float32)
                  + jnp.dot(pos8, W0p8, preferred_element_type=jnp.float32)
                  + fb0_ref[...])
    Q_ref[...] = jnp.dot(delta8 - pos8, W0p8,
                         preferred_element_type=jnp.float32)


def _stage1(x, pos8, h_W0, h_b0, hW18, hb18, W0x, W0p8, f_b0):
    blk = lambda i: (i, 0)
    full = lambda i: (0, 0)
    return pl.pallas_call(
        _stage1_body,
        grid=(N // _BN1,),
        in_specs=[
            pl.BlockSpec((_BN1, D), blk),
            pl.BlockSpec((_BN1, 8), blk),
            pl.BlockSpec((D, 64), full),
            pl.BlockSpec((1, 64), full),
            pl.BlockSpec((64, 8), full),
            pl.BlockSpec((1, 8), full),
            pl.BlockSpec((D, H), full),
            pl.BlockSpec((8, H), full),
            pl.BlockSpec((1, H), full),
        ],
        out_specs=[pl.BlockSpec((_BN1, H), blk), pl.BlockSpec((_BN1, H), blk)],
        out_shape=[jax.ShapeDtypeStruct((N, H), jnp.float32)] * 2,
    )(x, pos8, h_W0, h_b0, hW18, hb18, W0x, W0p8, f_b0)


_C2 = 64
_NCH2 = E // _C2


def _stage2(S, Q, src, dst):
    mesh = plsc.VectorSubcoreMesh(core_axis_name="c", subcore_axis_name="s")

    @functools.partial(
        pl.kernel, mesh=mesh,
        out_type=[jax.ShapeDtypeStruct((E, H), jnp.float32)] * 2,
        scratch_types=[
            pltpu.VMEM((_C2,), jnp.int32),
            pltpu.VMEM((_C2,), jnp.int32),
            pltpu.VMEM((_C2, H), jnp.float32),
            pltpu.VMEM((_C2, H), jnp.float32),
            pltpu.SemaphoreType.DMA,
        ],
    )
    def k(S_hbm, Q_hbm, src_hbm, dst_hbm, zs_hbm, zq_hbm,
          sidv, didv, bufS, bufQ, sem):
        wid = lax.axis_index("s") * _NC + lax.axis_index("c")
        nfull = _NCH2 // _NW
        nk = jnp.where(wid < _NCH2 - nfull * _NW, nfull + 1, nfull)

        def body(i, carry):
            base = (wid + i * _NW) * _C2
            pltpu.sync_copy(src_hbm.at[pl.ds(base, _C2)], sidv)
            pltpu.sync_copy(dst_hbm.at[pl.ds(base, _C2)], didv)
            cs = pltpu.async_copy(S_hbm.at[sidv], bufS, sem)
            cq = pltpu.async_copy(Q_hbm.at[didv], bufQ, sem)
            cs.wait()
            cq.wait()
            pltpu.sync_copy(bufS, zs_hbm.at[pl.ds(base, _C2)])
            pltpu.sync_copy(bufQ, zq_hbm.at[pl.ds(base, _C2)])
            return carry

        lax.fori_loop(0, nk, body, 0)

    return k(S, Q, src, dst)


_BE = 640
_NB = E // _BE


def _stage3_body(zs_ref, zq_ref, W1_ref, b1_ref, w_ref, m_ref):
    h1 = jnp.maximum(zs_ref[...] + zq_ref[...], 0.0)
    mm = jnp.dot(h1, W1_ref[...], preferred_element_type=jnp.float32) \
        + b1_ref[...]
    m_ref[...] = mm * w_ref[...]


def _stage3(zs, zq, f_W1, f_b1r, w2d):
    blk = lambda i: (i, 0)
    full = lambda i: (0, 0)
    return pl.pallas_call(
        _stage3_body,
        grid=(_NB,),
        in_specs=[
            pl.BlockSpec((_BE, H), blk),
            pl.BlockSpec((_BE, H), blk),
            pl.BlockSpec((H, H), full),
            pl.BlockSpec((1, H), full),
            pl.BlockSpec((_BE, 1), blk),
        ],
        out_specs=pl.BlockSpec((_BE, H), blk),
        out_shape=jax.ShapeDtypeStruct((E, H), jnp.float32),
    )(zs, zq, f_W1, f_b1r, w2d)


def _stage3a_body(dst_ref, rank_ref, hist_ref):
    d = dst_ref[...]
    owner = d // _OWN
    cols = lax.broadcasted_iota(jnp.int32, (_BE, _NW), 1)
    oh = jnp.where(owner == cols, 1.0, 0.0)
    ri = lax.broadcasted_iota(jnp.int32, (_BE, _BE), 0)
    ci = lax.broadcasted_iota(jnp.int32, (_BE, _BE), 1)
    ltri = jnp.where(ci < ri, 1.0, 0.0)
    rk = jnp.dot(ltri, oh, preferred_element_type=jnp.float32)
    rank = jnp.sum(oh * rk, axis=1, keepdims=True)
    rank_ref[...] = rank.astype(jnp.int32)
    hist_ref[...] = jnp.sum(oh, axis=0).reshape(1, 1, _NW)


def _stage3a(dst2d):
    return pl.pallas_call(
        _stage3a_body,
        grid=(_NB,),
        in_specs=[pl.BlockSpec((_BE, 1), lambda i: (i, 0))],
        out_specs=[pl.BlockSpec((_BE, 1), lambda i: (i, 0)),
                   pl.BlockSpec((1, 1, _NW), lambda i: (i, 0, 0))],
        out_shape=[jax.ShapeDtypeStruct((E, 1), jnp.int32),
                   jax.ShapeDtypeStruct((_NB, 1, _NW), jnp.float32)],
    )(dst2d)


def _stage3b_body(hist_ref, bases_ref, meta_ref):
    h = hist_ref[...].reshape(_NB, _NW)
    totals = jnp.sum(h, axis=0, keepdims=True)
    tp = jnp.floor((totals + 63.0) / 64.0) * 64.0
    ui = lax.broadcasted_iota(jnp.int32, (_NW, _NW), 0)
    uj = lax.broadcasted_iota(jnp.int32, (_NW, _NW), 1)
    ustri = jnp.where(ui < uj, 1.0, 0.0)
    owner_base = jnp.dot(tp, ustri,
                         preferred_element_type=jnp.float32)
    bi = lax.broadcasted_iota(jnp.int32, (_NB, _NB), 0)
    bj = lax.broadcasted_iota(jnp.int32, (_NB, _NB), 1)
    bltri = jnp.where(bj < bi, 1.0, 0.0)
    block_off = jnp.dot(bltri, h, preferred_element_type=jnp.float32)
    bases_ref[...] = (owner_base + block_off).astype(jnp.int32) \
        .reshape(_NB, 1, _NW)
    zeros14 = jnp.zeros((14, _NW), jnp.float32)
    meta_ref[...] = jnp.concatenate([owner_base, totals, zeros14],
                                    axis=0).astype(jnp.int32)


def _stage3b(hist):
    return pl.pallas_call(
        _stage3b_body,
        grid=(1,),
        in_specs=[pl.BlockSpec((_NB, 1, _NW), lambda i: (0, 0, 0))],
        out_specs=[pl.BlockSpec((_NB, 1, _NW), lambda i: (0, 0, 0)),
                   pl.BlockSpec((16, _NW), lambda i: (0, 0))],
        out_shape=[jax.ShapeDtypeStruct((_NB, 1, _NW), jnp.int32),
                   jax.ShapeDtypeStruct((16, _NW), jnp.int32)],
    )(hist)


def _stage3p_body(dst_ref, rank_ref, bases_ref, pos_ref):
    d = dst_ref[...]
    owner = d // _OWN
    cols = lax.broadcasted_iota(jnp.int32, (_BE, _NW), 1)
    b = bases_ref[...].reshape(1, _NW)
    sel = jnp.where(owner == cols, b, 0)
    pos_ref[...] = jnp.sum(sel, axis=1, keepdims=True) + rank_ref[...]


def _stage3p(dst2d, rank, bases):
    return pl.pallas_call(
        _stage3p_body,
        grid=(_NB,),
        in_specs=[
            pl.BlockSpec((_BE, 1), lambda i: (i, 0)),
            pl.BlockSpec((_BE, 1), lambda i: (i, 0)),
            pl.BlockSpec((1, 1, _NW), lambda i: (i, 0, 0)),
        ],
        out_specs=pl.BlockSpec((_BE, 1), lambda i: (i, 0)),
        out_shape=jax.ShapeDtypeStruct((E, 1), jnp.int32),
    )(dst2d, rank, bases)


_C3 = 128
_NCH3 = E // _C3


def _stage3c(pos, dst):
    mesh = plsc.VectorSubcoreMesh(core_axis_name="c", subcore_axis_name="s")

    @functools.partial(
        pl.kernel, mesh=mesh,
        out_type=[jax.ShapeDtypeStruct((_EPAD,), jnp.int32)] * 2,
        scratch_types=[
            pltpu.VMEM((_C3,), jnp.int32),
            pltpu.VMEM((_C3,), jnp.int32),
            pltpu.VMEM((_C3,), jnp.int32),
            pltpu.SemaphoreType.DMA,
        ],
    )
    def k(pos_hbm, dst_hbm, eidb_hbm, dstb_hbm, posv, eidv, dstv, sem):
        wid = lax.axis_index("s") * _NC + lax.axis_index("c")
        iota = lax.iota(jnp.int32, 16)
        nfull = _NCH3 // _NW
        nk = jnp.where(wid < _NCH3 - nfull * _NW, nfull + 1, nfull)

        def body(i, carry):
            base = (wid + i * _NW) * _C3
            pltpu.sync_copy(pos_hbm.at[pl.ds(base, _C3)], posv)
            pltpu.sync_copy(dst_hbm.at[pl.ds(base, _C3)], dstv)
            for t in range(_C3 // 16):
                eidv[pl.ds(t * 16, 16)] = base + t * 16 + iota
            c1 = pltpu.async_copy(eidv, eidb_hbm.at[posv], sem)
            c2 = pltpu.async_copy(dstv, dstb_hbm.at[posv], sem)
            c1.wait()
            c2.wait()
            return carry

        lax.fori_loop(0, nk, body, 0)

    return k(pos, dst)


_K4 = 64


def _stage4(m2, eidb, dstb, meta):
    mesh = plsc.VectorSubcoreMesh(core_axis_name="c", subcore_axis_name="s")

    @functools.partial(
        pl.kernel, mesh=mesh,
        out_type=jax.ShapeDtypeStruct((2, _NPAD * _HH), jnp.float32),
        scratch_types=[
            pltpu.VMEM((_OWNP * _HH,), jnp.float32),
            pltpu.VMEM((16 * _NW,), jnp.int32),
            pltpu.VMEM((_K4,), jnp.int32),
            pltpu.VMEM((_K4,), jnp.int32),
            pltpu.VMEM((_K4,), jnp.int32),
            pltpu.VMEM((_K4 + 16,), jnp.int32),
            pltpu.VMEM((_K4, _HH), jnp.float32),
            pltpu.SemaphoreType.DMA,
        ],
    )
    def k(m_hbm, eidb_hbm, dstb_hbm, meta_hbm, agg_hbm,
          accf, metav, eidv, dstv, gidx, gloc, rows, sem):
        wid = lax.axis_index("s") * _NC + lax.axis_index("c")
        lo = wid * _OWN
        neg = jnp.full((16,), _NEG, jnp.float32)
        iota = lax.iota(jnp.int32, 16)
        zero16 = jnp.zeros((16,), jnp.int32)

        pltpu.sync_copy(meta_hbm, metav)

        def pick(row):
            a = metav[pl.ds(row * _NW, 16)]
            b = metav[pl.ds(row * _NW + 16, 16)]
            s = jnp.where(iota == wid, a, zero16) \
                + jnp.where(iota == wid - 16, b, zero16)
            for sh in (8, 4, 2, 1):
                s = s + s.at[iota ^ sh].get(mode="promise_in_bounds")
            return s[0]

        start = pl.multiple_of(pick(0), _K4)
        cnt = pick(1)
        nchunk = (cnt + (_K4 - 1)) // _K4

        def init_acc(i, c):
            accf[pl.ds(i * 16, 16)] = neg
            return c

        for p in (0, 1):
            lax.fori_loop(0, (_OWNP * _HH) // 16, init_acc, 0)

            def chunk(j, carry):
                off = start + j * _K4
                pltpu.sync_copy(eidb_hbm.at[pl.ds(off, _K4)], eidv)
                pltpu.sync_copy(dstb_hbm.at[pl.ds(off, _K4)], dstv)
                for t in range(_K4 // 16):
                    e = eidv[pl.ds(t * 16, 16)]
                    e = jnp.minimum(jnp.maximum(e, 0), E - 1)
                    gidx[pl.ds(t * 16, 16)] = e * 2 + p
                    d = dstv[pl.ds(t * 16, 16)]
                    l = d - lo
                    okm = (l >= 0) & (l < _OWN)
                    gloc[pl.ds(t * 16, 16)] = jnp.where(
                        okm, l, jnp.full((16,), _OWN, jnp.int32))
                pltpu.async_copy(m_hbm.at[gidx], rows, sem).wait()

                def maxgrp(g, c):
                    gbase = pl.multiple_of(g * 16, 16)
                    lv = gloc[pl.ds(gbase, 16)]
                    for lane in range(16):
                        l = lv[lane]
                        rb = pl.multiple_of(l * _HH, _HH)
                        r = gbase + lane
                        for t in range(_HH // 16):
                            a = accf[pl.ds(rb + t * 16, 16)]
                            b = rows[r, pl.ds(t * 16, 16)]
                            accf[pl.ds(rb + t * 16, 16)] = jnp.maximum(a, b)
                    return c
                lax.fori_loop(0, _K4 // 16, maxgrp, 0)
                return carry

            lax.fori_loop(0, nchunk, chunk, 0)

            pltpu.sync_copy(accf.at[pl.ds(0, _OWN * _HH)],
                            agg_hbm.at[p, pl.ds(lo * _HH, _OWN * _HH)])

    return k(m2, eidb, dstb, meta)


_BN5 = 1000


def _stage5a_body(a_ref, x_ref, gW0_ref, gb0_ref, gW1_ref, gb1_ref,
                  y_ref, st_ref):
    i = pl.program_id(0)
    g = jnp.concatenate([a_ref[0], a_ref[1]], axis=-1)
    g = jnp.where(g <= -1.0e38, 0.0, g)
    hid = jnp.maximum(
        jnp.dot(g, gW0_ref[...], preferred_element_type=jnp.float32)
        + gb0_ref[...], 0.0)
    out = jnp.dot(hid, gW1_ref[...], preferred_element_type=jnp.float32) \
        + gb1_ref[...]
    y = x_ref[...] + out
    y_ref[...] = y

    @pl.when(i == 0)
    def _():
        st_ref[...] = jnp.zeros_like(st_ref)

    st_ref[0:1, :] += jnp.sum(y, axis=0, keepdims=True)
    st_ref[1:2, :] += jnp.sum(y * y, axis=0, keepdims=True)


def _stage5a(agg2, x, g_W0, g_b0r, g_W1, g_b1r):
    return pl.pallas_call(
        _stage5a_body,
        grid=(N // _BN5,),
        in_specs=[
            pl.BlockSpec((2, _BN5, _HH), lambda i: (0, i, 0)),
            pl.BlockSpec((_BN5, D), lambda i: (i, 0)),
            pl.BlockSpec((H, H), lambda i: (0, 0)),
            pl.BlockSpec((1, H), lambda i: (0, 0)),
            pl.BlockSpec((H, D), lambda i: (0, 0)),
            pl.BlockSpec((1, D), lambda i: (0, 0)),
        ],
        out_specs=[pl.BlockSpec((_BN5, D), lambda i: (i, 0)),
                   pl.BlockSpec((8, D), lambda i: (0, 0))],
        out_shape=[jax.ShapeDtypeStruct((N, D), jnp.float32),
                   jax.ShapeDtypeStruct((8, D), jnp.float32)],
    )(agg2, x, g_W0, g_b0r, g_W1, g_b1r)


def _stage5b_body(y_ref, st_ref, bng_ref, bnb_ref, o_ref):
    mean = st_ref[0:1, :] / N
    var = st_ref[1:2, :] / N - mean * mean
    inv = lax.rsqrt(var + 1e-5)
    yn = (y_ref[...] - mean) * inv * bng_ref[...] + bnb_ref[...]
    o_ref[...] = jnp.maximum(yn, 0.0)


def _stage5b(y, st, bngr, bnbr):
    return pl.pallas_call(
        _stage5b_body,
        grid=(N // _BN5,),
        in_specs=[
            pl.BlockSpec((_BN5, D), lambda i: (i, 0)),
            pl.BlockSpec((8, D), lambda i: (0, 0)),
            pl.BlockSpec((1, D), lambda i: (0, 0)),
            pl.BlockSpec((1, D), lambda i: (0, 0)),
        ],
        out_specs=pl.BlockSpec((_BN5, D), lambda i: (i, 0)),
        out_shape=jax.ShapeDtypeStruct((N, D), jnp.float32),
    )(y, st, bngr, bnbr)


def kernel(x, pos, edge_index, edge_weight,
           h_W0, h_b0, h_W1, h_b1,
           f_W0, f_b0, f_W1, f_b1,
           g_W0, g_b0, g_W1, g_b1,
           bn_g, bn_b):
    src = edge_index[0]
    dst = edge_index[1]

    pos8 = jnp.pad(pos, ((0, 0), (0, 5)))
    W0p8 = jnp.pad(f_W0[:3], ((0, 5), (0, 0)))
    W0x = f_W0[3:]
    hW18 = jnp.pad(h_W1, ((0, 0), (0, 5)))
    hb18 = jnp.pad(h_b1, (0, 5)).reshape(1, 8)

    S, Q = _stage1(x, pos8, h_W0, h_b0.reshape(1, 64), hW18, hb18,
                   W0x, W0p8, f_b0.reshape(1, H))

    zs, zq = _stage2(S, Q, src, dst)

    m = _stage3(zs, zq, f_W1, f_b1.reshape(1, H),
                edge_weight.reshape(E, 1))

    dst2d = dst.reshape(E, 1)
    rank, hist = _stage3a(dst2d)
    bases, meta = _stage3b(hist)
    pos_ = _stage3p(dst2d, rank, bases).reshape(E)
    eidb, dstb = _stage3c(pos_, dst)

    m2 = m.reshape(E * 2, _HH)
    agg2 = _stage4(m2, eidb, dstb, meta.reshape(16 * _NW)) \
        .reshape(2, _NPAD, _HH)

    y, st = _stage5a(agg2, x, g_W0, g_b0.reshape(1, H),
                     g_W1, g_b1.reshape(1, D))
    return _stage5b(y, st, bn_g.reshape(1, D), bn_b.reshape(1, D))


# speedup vs baseline: 1.2074x; 1.0842x over previous
import functools

import jax
import jax.numpy as jnp
from jax import lax
from jax.experimental import pallas as pl
from jax.experimental.pallas import tpu as pltpu
from jax.experimental.pallas import tpu_sc as plsc

N = 10000
E = 160000
D = 256
H = 512

_NC = 2
_NS = 16
_NW = _NC * _NS

_OWN = 313
_NPAD = _NW * _OWN
_OWNP = 320
_NEG = -3.0e38
_EPAD = E + _NW * 64 + 512

_BN1 = 1000


def _stage1_body(x_ref, pos8_ref, hW0_ref, hb0_ref, hW18_ref, hb18_ref,
                 W0x_ref, W0p8_ref, fb0_ref, S_ref, Q_ref):
    x = x_ref[...]
    pos8 = pos8_ref[...]
    hh = jnp.maximum(
        jnp.dot(x, hW0_ref[...], preferred_element_type=jnp.float32)
        + hb0_ref[...], 0.0)
    delta8 = jnp.dot(hh, hW18_ref[...], preferred_element_type=jnp.float32) \
        + hb18_ref[...]
    W0p8 = W0p8_ref[...]
    S = (jnp.dot(x, W0x_ref[...], preferred_element_type=jnp.float32)
         + jnp.dot(pos8, W0p8, preferred_element_type=jnp.float32)
         + fb0_ref[...])
    Q = jnp.dot(delta8 - pos8, W0p8, preferred_element_type=jnp.float32)
    S_ref[...] = S
    Q_ref[...] = Q


def _stage1(x, pos8, h_W0, h_b0, hW18, hb18, W0x, W0p8, f_b0):
    blk = lambda i: (i, 0)
    full = lambda i: (0, 0)
    return pl.pallas_call(
        _stage1_body,
        grid=(N // _BN1,),
        in_specs=[
            pl.BlockSpec((_BN1, D), blk),
            pl.BlockSpec((_BN1, 8), blk),
            pl.BlockSpec((D, 64), full),
            pl.BlockSpec((1, 64), full),
            pl.BlockSpec((64, 8), full),
            pl.BlockSpec((1, 8), full),
            pl.BlockSpec((D, H), full),
            pl.BlockSpec((8, H), full),
            pl.BlockSpec((1, H), full),
        ],
        out_specs=[pl.BlockSpec((_BN1, H), blk), pl.BlockSpec((_BN1, H), blk)],
        out_shape=[jax.ShapeDtypeStruct((N, H), jnp.float32)] * 2,
    )(x, pos8, h_W0, h_b0, hW18, hb18, W0x, W0p8, f_b0)


_C2 = 64
_NCH2 = E // _C2


def _stage2(S, Q, src, dst):
    mesh = plsc.VectorSubcoreMesh(core_axis_name="c", subcore_axis_name="s")

    @functools.partial(
        pl.kernel, mesh=mesh,
        out_type=[jax.ShapeDtypeStruct((E, H), jnp.float32)] * 2,
        scratch_types=[
            pltpu.VMEM((_C2,), jnp.int32),
            pltpu.VMEM((_C2,), jnp.int32),
            pltpu.VMEM((_C2, H), jnp.float32),
            pltpu.VMEM((_C2, H), jnp.float32),
            pltpu.SemaphoreType.DMA,
        ],
    )
    def k(S_hbm, Q_hbm, src_hbm, dst_hbm, zs_hbm, zq_hbm,
          sidv, didv, bufS, bufQ, sem):
        wid = lax.axis_index("s") * _NC + lax.axis_index("c")
        nfull = _NCH2 // _NW
        nk = jnp.where(wid < _NCH2 - nfull * _NW, nfull + 1, nfull)

        def body(i, carry):
            base = (wid + i * _NW) * _C2
            pltpu.sync_copy(src_hbm.at[pl.ds(base, _C2)], sidv)
            pltpu.sync_copy(dst_hbm.at[pl.ds(base, _C2)], didv)
            cs = pltpu.async_copy(S_hbm.at[sidv], bufS, sem)
            cq = pltpu.async_copy(Q_hbm.at[didv], bufQ, sem)
            cs.wait()
            cq.wait()
            pltpu.sync_copy(bufS, zs_hbm.at[pl.ds(base, _C2)])
            pltpu.sync_copy(bufQ, zq_hbm.at[pl.ds(base, _C2)])
            return carry

        lax.fori_loop(0, nk, body, 0)

    return k(S, Q, src, dst)


_BE = 640
_NB = E // _BE


def _stage3_body(zs_ref, zq_ref, W1_ref, b1_ref, w_ref, m_ref):
    z = zs_ref[...] + zq_ref[...]
    h1 = jnp.maximum(z, 0.0).astype(jnp.bfloat16)
    mm = jnp.dot(h1, W1_ref[...], preferred_element_type=jnp.float32) \
        + b1_ref[...]
    m_ref[...] = mm * w_ref[...]


def _stage3(zs, zq, f_W1bf, f_b1r, w2d):
    blk = lambda i: (i, 0)
    full = lambda i: (0, 0)
    return pl.pallas_call(
        _stage3_body,
        grid=(_NB,),
        in_specs=[
            pl.BlockSpec((_BE, H), blk),
            pl.BlockSpec((_BE, H), blk),
            pl.BlockSpec((H, H), full),
            pl.BlockSpec((1, H), full),
            pl.BlockSpec((_BE, 1), blk),
        ],
        out_specs=pl.BlockSpec((_BE, H), blk),
        out_shape=jax.ShapeDtypeStruct((E, H), jnp.float32),
    )(zs, zq, f_W1bf, f_b1r, w2d)


def _stage3a_body(dst_ref, rank_ref, hist_ref):
    d = dst_ref[...]
    owner = d // _OWN
    cols = lax.broadcasted_iota(jnp.int32, (_BE, _NW), 1)
    oh = jnp.where(owner == cols, 1.0, 0.0)
    ri = lax.broadcasted_iota(jnp.int32, (_BE, _BE), 0)
    ci = lax.broadcasted_iota(jnp.int32, (_BE, _BE), 1)
    ltri = jnp.where(ci < ri, 1.0, 0.0)
    rk = jnp.dot(ltri, oh, preferred_element_type=jnp.float32)
    rank = jnp.sum(oh * rk, axis=1, keepdims=True)
    rank_ref[...] = rank.astype(jnp.int32)
    hist_ref[...] = jnp.sum(oh, axis=0).reshape(1, 1, _NW)


def _stage3a(dst2d):
    return pl.pallas_call(
        _stage3a_body,
        grid=(_NB,),
        in_specs=[pl.BlockSpec((_BE, 1), lambda i: (i, 0))],
        out_specs=[pl.BlockSpec((_BE, 1), lambda i: (i, 0)),
                   pl.BlockSpec((1, 1, _NW), lambda i: (i, 0, 0))],
        out_shape=[jax.ShapeDtypeStruct((E, 1), jnp.int32),
                   jax.ShapeDtypeStruct((_NB, 1, _NW), jnp.float32)],
    )(dst2d)


def _stage3b_body(hist_ref, bases_ref, meta_ref):
    h = hist_ref[...].reshape(_NB, _NW)
    totals = jnp.sum(h, axis=0, keepdims=True)
    tp = jnp.floor((totals + 63.0) / 64.0) * 64.0
    ui = lax.broadcasted_iota(jnp.int32, (_NW, _NW), 0)
    uj = lax.broadcasted_iota(jnp.int32, (_NW, _NW), 1)
    ustri = jnp.where(ui < uj, 1.0, 0.0)
    owner_base = jnp.dot(tp, ustri,
                         preferred_element_type=jnp.float32)
    bi = lax.broadcasted_iota(jnp.int32, (_NB, _NB), 0)
    bj = lax.broadcasted_iota(jnp.int32, (_NB, _NB), 1)
    bltri = jnp.where(bj < bi, 1.0, 0.0)
    block_off = jnp.dot(bltri, h, preferred_element_type=jnp.float32)
    bases_ref[...] = (owner_base + block_off).astype(jnp.int32) \
        .reshape(_NB, 1, _NW)
    zeros14 = jnp.zeros((14, _NW), jnp.float32)
    meta_ref[...] = jnp.concatenate([owner_base, totals, zeros14],
                                    axis=0).astype(jnp.int32)


def _stage3b(hist):
    return pl.pallas_call(
        _stage3b_body,
        grid=(1,),
        in_specs=[pl.BlockSpec((_NB, 1, _NW), lambda i: (0, 0, 0))],
        out_specs=[pl.BlockSpec((_NB, 1, _NW), lambda i: (0, 0, 0)),
                   pl.BlockSpec((16, _NW), lambda i: (0, 0))],
        out_shape=[jax.ShapeDtypeStruct((_NB, 1, _NW), jnp.int32),
                   jax.ShapeDtypeStruct((16, _NW), jnp.int32)],
    )(hist)


def _stage3p_body(dst_ref, rank_ref, bases_ref, pos_ref, pak_ref):
    i = pl.program_id(0)
    d = dst_ref[...]
    owner = d // _OWN
    cols = lax.broadcasted_iota(jnp.int32, (_BE, _NW), 1)
    b = bases_ref[...].reshape(1, _NW)
    sel = jnp.where(owner == cols, b, 0)
    pos_ref[...] = jnp.sum(sel, axis=1, keepdims=True) + rank_ref[...]
    eid = lax.broadcasted_iota(jnp.int32, (_BE, 1), 0) + i * _BE
    pak_ref[...] = eid * 512 + (d - owner * _OWN)


def _stage3p(dst2d, rank, bases):
    return pl.pallas_call(
        _stage3p_body,
        grid=(_NB,),
        in_specs=[
            pl.BlockSpec((_BE, 1), lambda i: (i, 0)),
            pl.BlockSpec((_BE, 1), lambda i: (i, 0)),
            pl.BlockSpec((1, 1, _NW), lambda i: (i, 0, 0)),
        ],
        out_specs=[pl.BlockSpec((_BE, 1), lambda i: (i, 0)),
                   pl.BlockSpec((_BE, 1), lambda i: (i, 0))],
        out_shape=[jax.ShapeDtypeStruct((E, 1), jnp.int32),
                   jax.ShapeDtypeStruct((E, 1), jnp.int32)],
    )(dst2d, rank, bases)


_C3 = 128
_NR3 = E // _C3
_G3 = 8
_NG3 = _NR3 // _G3


def _stage3c(pos2, pak2):
    mesh = plsc.VectorSubcoreMesh(core_axis_name="c", subcore_axis_name="s")

    @functools.partial(
        pl.kernel, mesh=mesh,
        out_type=jax.ShapeDtypeStruct((_EPAD,), jnp.int32),
        scratch_types=[
            pltpu.VMEM((_G3, _C3), jnp.int32),
            pltpu.VMEM((_G3, _C3), jnp.int32),
            pltpu.SemaphoreType.DMA,
            pltpu.SemaphoreType.DMA,
        ],
    )
    def k(pos_hbm, pak_hbm, binned_hbm, posv, pakv, seml, sems):
        wid = lax.axis_index("s") * _NC + lax.axis_index("c")
        nfull = _NG3 // _NW
        nk = jnp.where(wid < _NG3 - nfull * _NW, nfull + 1, nfull)

        def body(i, carry):
            g = wid + i * _NW
            cl = pltpu.async_copy(pos_hbm.at[pl.ds(g * _G3, _G3)], posv, seml)
            cp = pltpu.async_copy(pak_hbm.at[pl.ds(g * _G3, _G3)], pakv, seml)
            cl.wait()
            cp.wait()
            cs = [pltpu.async_copy(pakv.at[j], binned_hbm.at[posv.at[j]],
                                   sems) for j in range(_G3)]
            for c in cs:
                c.wait()
            return carry

        lax.fori_loop(0, nk, body, 0)

        @pl.when(wid == 0)
        def _():
            cl = pltpu.async_copy(pos_hbm.at[pl.ds(_NG3 * _G3, 2)],
                                  posv.at[pl.ds(0, 2)], seml)
            cp = pltpu.async_copy(pak_hbm.at[pl.ds(_NG3 * _G3, 2)],
                                  pakv.at[pl.ds(0, 2)], seml)
            cl.wait()
            cp.wait()
            for j in range(2):
                pltpu.async_copy(pakv.at[j], binned_hbm.at[posv.at[j]],
                                 sems).wait()

    return k(pos2, pak2)


_K4 = 64
_GR4 = 8


_HH = H // 2


def _stage4(m2, binned, meta):
    mesh = plsc.VectorSubcoreMesh(core_axis_name="c", subcore_axis_name="s")

    @functools.partial(
        pl.kernel, mesh=mesh,
        out_type=jax.ShapeDtypeStruct((2, _NPAD * _HH), jnp.float32),
        scratch_types=[
            pltpu.VMEM((_OWNP * _HH,), jnp.float32),
            pltpu.VMEM((16 * _NW,), jnp.int32),
            pltpu.VMEM((_K4 * _GR4,), jnp.int32),
            pltpu.VMEM((_K4,), jnp.int32),
            pltpu.VMEM((_K4,), jnp.int32),
            pltpu.VMEM((_K4 + 16,), jnp.int32),
            pltpu.VMEM((_K4 + 16,), jnp.int32),
            pltpu.VMEM((_K4, _HH), jnp.float32),
            pltpu.VMEM((_K4, _HH), jnp.float32),
            pltpu.SemaphoreType.DMA,
            pltpu.SemaphoreType.DMA,
            pltpu.SemaphoreType.DMA,
        ],
    )
    def k(m_hbm, binned_hbm, meta_hbm, agg_hbm,
          accf, metav, pkv, gidx0, gidx1, gloc0, gloc1, rows0, rows1,
          semi, sem0, sem1):
        wid = lax.axis_index("s") * _NC + lax.axis_index("c")
        lo = wid * _OWN
        neg = jnp.full((16,), _NEG, jnp.float32)
        iota = lax.iota(jnp.int32, 16)
        zero16 = jnp.zeros((16,), jnp.int32)
        own16 = jnp.full((16,), _OWN, jnp.int32)

        pltpu.sync_copy(meta_hbm, metav)

        def pick(row):
            a = metav[pl.ds(row * _NW, 16)]
            b = metav[pl.ds(row * _NW + 16, 16)]
            s = jnp.where(iota == wid, a, zero16) \
                + jnp.where(iota == wid - 16, b, zero16)
            for sh in (8, 4, 2, 1):
                s = s + s.at[iota ^ sh].get(mode="promise_in_bounds")
            return s[0]

        start = pl.multiple_of(pick(0), _K4)
        cnt = pick(1)
        ngrp = (cnt + (_K4 * _GR4 - 1)) // (_K4 * _GR4)

        def init_acc(i, c):
            accf[pl.ds(i * 16, 16)] = neg
            return c

        gidx = (gidx0, gidx1)
        gloc = (gloc0, gloc1)
        rows = (rows0, rows1)
        sems = (sem0, sem1)

        def half(p, hcarry):
            lax.fori_loop(0, (_OWNP * _HH) // 16, init_acc, 0)

            def fire(jj, valid_base):
                b = gidx[jj % 2]
                lv = gloc[jj % 2]
                for t in range(_K4 // 16):
                    pk = pkv[pl.ds(jj * _K4 + t * 16, 16)]
                    e = jnp.minimum(jnp.maximum(pk >> 9, 0), E - 1)
                    l = pk & 511
                    idx = valid_base + jj * _K4 + t * 16 + iota
                    okm = idx < cnt
                    b[pl.ds(t * 16, 16)] = e * 2 + p
                    lv[pl.ds(t * 16, 16)] = jnp.where(
                        okm, jnp.minimum(l, own16), own16)
                return pltpu.async_copy(m_hbm.at[b], rows[jj % 2],
                                        sems[jj % 2])

            def rmw(jj):
                lvr = gloc[jj % 2]
                rr = rows[jj % 2]

                def maxgrp(g, c):
                    gbase = pl.multiple_of(g * 16, 16)
                    lv = lvr[pl.ds(gbase, 16)]
                    for lane in range(16):
                        l = lv[lane]
                        rb = pl.multiple_of(l * _HH, _HH)
                        r = gbase + lane

                        def tbody(t, c2):
                            tb = pl.multiple_of(t * 32, 32)
                            for u in (0, 16):
                                a = accf[pl.ds(rb + tb + u, 16)]
                                bb = rr[r, pl.ds(tb + u, 16)]
                                accf[pl.ds(rb + tb + u, 16)] = \
                                    jnp.maximum(a, bb)
                            return c2
                        lax.fori_loop(0, _HH // 32, tbody, 0)
                    return c
                lax.fori_loop(0, _K4 // 16, maxgrp, 0)

            def group(jg, carry):
                off = start + jg * (_K4 * _GR4)
                pltpu.sync_copy(binned_hbm.at[pl.ds(off, _K4 * _GR4)], pkv)
                vb = jg * (_K4 * _GR4)
                c = fire(0, vb)
                for jj in range(_GR4):
                    c_next = fire(jj + 1, vb) if jj + 1 < _GR4 else None
                    c.wait()
                    rmw(jj)
                    c = c_next
                return carry

            lax.fori_loop(0, ngrp, group, 0)

            pltpu.sync_copy(accf.at[pl.ds(0, _OWN * _HH)],
                            agg_hbm.at[p, pl.ds(lo * _HH, _OWN * _HH)])
            return hcarry

        lax.fori_loop(0, 2, half, 0)

    return k(m2, binned, meta)


_BN5 = 1000


def _stage5a_body(a_ref, x_ref, gW0_ref, gb0_ref, gW1_ref, gb1_ref,
                  y_ref, st_ref):
    i = pl.program_id(0)
    g = jnp.concatenate([a_ref[0], a_ref[1]], axis=-1)
    g = jnp.where(g <= -1.0e38, 0.0, g)
    hid = jnp.maximum(
        jnp.dot(g, gW0_ref[...], preferred_element_type=jnp.float32)
        + gb0_ref[...], 0.0)
    out = jnp.dot(hid, gW1_ref[...], preferred_element_type=jnp.float32) \
        + gb1_ref[...]
    y = x_ref[...] + out
    y_ref[...] = y

    @pl.when(i == 0)
    def _():
        st_ref[...] = jnp.zeros_like(st_ref)

    st_ref[0:1, :] += jnp.sum(y, axis=0, keepdims=True)
    st_ref[1:2, :] += jnp.sum(y * y, axis=0, keepdims=True)


def _stage5a(agg2, x, g_W0, g_b0r, g_W1, g_b1r):
    return pl.pallas_call(
        _stage5a_body,
        grid=(N // _BN5,),
        in_specs=[
            pl.BlockSpec((2, _BN5, _HH), lambda i: (0, i, 0)),
            pl.BlockSpec((_BN5, D), lambda i: (i, 0)),
            pl.BlockSpec((H, H), lambda i: (0, 0)),
            pl.BlockSpec((1, H), lambda i: (0, 0)),
            pl.BlockSpec((H, D), lambda i: (0, 0)),
            pl.BlockSpec((1, D), lambda i: (0, 0)),
        ],
        out_specs=[pl.BlockSpec((_BN5, D), lambda i: (i, 0)),
                   pl.BlockSpec((8, D), lambda i: (0, 0))],
        out_shape=[jax.ShapeDtypeStruct((N, D), jnp.float32),
                   jax.ShapeDtypeStruct((8, D), jnp.float32)],
    )(agg2, x, g_W0, g_b0r, g_W1, g_b1r)


def _stage5b_body(y_ref, st_ref, bng_ref, bnb_ref, o_ref):
    mean = st_ref[0:1, :] / N
    var = st_ref[1:2, :] / N - mean * mean
    inv = lax.rsqrt(var + 1e-5)
    yn = (y_ref[...] - mean) * inv * bng_ref[...] + bnb_ref[...]
    o_ref[...] = jnp.maximum(yn, 0.0)


def _stage5b(y, st, bngr, bnbr):
    return pl.pallas_call(
        _stage5b_body,
        grid=(N // _BN5,),
        in_specs=[
            pl.BlockSpec((_BN5, D), lambda i: (i, 0)),
            pl.BlockSpec((8, D), lambda i: (0, 0)),
            pl.BlockSpec((1, D), lambda i: (0, 0)),
            pl.BlockSpec((1, D), lambda i: (0, 0)),
        ],
        out_specs=pl.BlockSpec((_BN5, D), lambda i: (i, 0)),
        out_shape=jax.ShapeDtypeStruct((N, D), jnp.float32),
    )(y, st, bngr, bnbr)


def kernel(x, pos, edge_index, edge_weight,
           h_W0, h_b0, h_W1, h_b1,
           f_W0, f_b0, f_W1, f_b1,
           g_W0, g_b0, g_W1, g_b1,
           bn_g, bn_b):
    src = edge_index[0]
    dst = edge_index[1]

    pos8 = jnp.pad(pos, ((0, 0), (0, 5)))
    W0p8 = jnp.pad(f_W0[:3], ((0, 5), (0, 0)))
    W0x = f_W0[3:]
    hW18 = jnp.pad(h_W1, ((0, 0), (0, 5)))
    hb18 = jnp.pad(h_b1, (0, 5)).reshape(1, 8)

    S, Q = _stage1(x, pos8, h_W0, h_b0.reshape(1, 64), hW18, hb18,
                   W0x, W0p8, f_b0.reshape(1, H))

    zs, zq = _stage2(S, Q, src, dst)

    m = _stage3(zs, zq, f_W1.astype(jnp.bfloat16), f_b1.reshape(1, H),
                edge_weight.reshape(E, 1))

    dst2d = dst.reshape(E, 1)
    rank, hist = _stage3a(dst2d)
    bases, meta = _stage3b(hist)
    pos_, pak = _stage3p(dst2d, rank, bases)
    binned = _stage3c(pos_.reshape(_NR3, _C3), pak.reshape(_NR3, _C3))

    m2 = m.reshape(E * 2, _HH)
    agg2 = _stage4(m2, binned, meta.reshape(16 * _NW)) \
        .reshape(2, _NPAD, _HH)

    y, st = _stage5a(agg2, x, g_W0, g_b0.reshape(1, H),
                     g_W1, g_b1.reshape(1, D))
    return _stage5b(y, st, bn_g.reshape(1, D), bn_b.reshape(1, D))
